# baseline (speedup 1.0000x reference)
import functools

import jax
import jax.numpy as jnp
from jax import lax
from jax.experimental import pallas as pl
from jax.experimental.pallas import tpu as pltpu
from jax.experimental.pallas import tpu_sc as plsc

D = 32
SEQ = 200
BB = 128
TC = 8
NTC = SEQ // TC
VL = 16


def kernel(inputs, token_table, pos_table):
    B, S = inputs.shape
    V, d = token_table.shape
    assert S == SEQ and d == D and B % BB == 0

    info = plsc.get_sparse_core_info()
    NC, NS = info.num_cores, info.num_subcores
    NW = NC * NS
    assert B // BB == NW

    idx_i32 = inputs.astype(jnp.int32)

    mesh = plsc.VectorSubcoreMesh(core_axis_name="c", subcore_axis_name="s")

    @functools.partial(
        pl.kernel,
        mesh=mesh,
        compiler_params=pltpu.CompilerParams(use_tc_tiling_on_sc=False, needs_layout_passes=False),
        out_type=jax.ShapeDtypeStruct((SEQ, D // 8, NW, 8, BB), jnp.float32),
        scratch_types=[
            pltpu.VMEM((BB, SEQ), jnp.int32),
            pltpu.VMEM((2, TC, BB), jnp.int32),
            pltpu.VMEM((2, TC, BB, D), jnp.float32),
            pltpu.VMEM((2, D // 8, 8, BB), jnp.float32),
            pltpu.VMEM((SEQ, D), jnp.float32),
            [pltpu.SemaphoreType.DMA] * 2,
            [pltpu.SemaphoreType.DMA] * 2,
        ],
    )
    def body(idx_hbm, tok_hbm, pos_hbm, out_hbm,
             idx_all, idx_c, gbuf, stage, pos_v, gsems, wsems):
        wid = lax.axis_index("s") * NC + lax.axis_index("c")
        pltpu.sync_copy(pos_hbm, pos_v)
        pltpu.sync_copy(idx_hbm.at[pl.ds(wid * BB, BB)], idx_all)

        lanes = lax.iota(jnp.int32, VL)

        def build_and_fire(tc, par):
            for tl in range(TC):
                t = tc * TC + tl
                tvec = jnp.full((VL,), 0, jnp.int32) + t
                for v in range(BB // VL):
                    vals = plsc.load_gather(idx_all, [lanes + (v * VL), tvec])
                    idx_c[par, tl, pl.ds(v * VL, VL)] = vals
            for tl in range(TC):
                pltpu.async_copy(
                    tok_hbm.at[idx_c.at[par, tl]],
                    gbuf.at[par, tl],
                    gsems[par],
                )

        def drain_gather(par):
            for tl in range(TC):
                pltpu.make_async_copy(
                    tok_hbm.at[pl.ds(0, BB)], gbuf.at[par, tl], gsems[par]
                ).wait()

        def process_chunk(tc, par):
            def do_t(tl, st, first_ok):
                t = tc * TC + tl

                @pl.when(first_ok)
                def _():
                    for a in range(D // 8):
                        pltpu.make_async_copy(
                            stage.at[st, a], out_hbm.at[0, a, wid], wsems[st]
                        ).wait()

                p0 = pos_v[t, pl.ds(0, VL)]
                p1 = pos_v[t, pl.ds(VL, VL)]
                for a in range(D // 8):
                    for cl in range(8):
                        c = a * 8 + cl
                        cvec = jnp.full((VL,), 0, jnp.int32) + c
                        pv = (p0 if c < VL else p1)[c % VL]
                        for v in range(BB // VL):
                            g = plsc.load_gather(
                                gbuf,
                                [
                                    jnp.full((VL,), 0, jnp.int32) + par,
                                    jnp.full((VL,), 0, jnp.int32) + tl,
                                    lanes + (v * VL),
                                    cvec,
                                ],
                            )
                            stage[st, a, cl, pl.ds(v * VL, VL)] = g + pv
                for a in range(D // 8):
                    pltpu.async_copy(
                        stage.at[st, a], out_hbm.at[t, a, wid], wsems[st]
                    )

            def two_t(tl2, _):
                ok = (tc > 0) | (tl2 >= 1)
                for par_t in range(2):
                    do_t(tl2 * 2 + par_t, par_t, ok)
                return 0

            lax.fori_loop(0, TC // 2, two_t, 0)

        build_and_fire(0, 0)

        def pair(p, _):
            c0 = 2 * p
            c1 = c0 + 1

            @pl.when(c1 < NTC)
            def _():
                build_and_fire(c1, 1)

            drain_gather(0)
            process_chunk(c0, 0)

            @pl.when(c1 < NTC)
            def _():
                @pl.when(c1 + 1 < NTC)
                def _():
                    build_and_fire(c1 + 1, 0)

                drain_gather(1)
                process_chunk(c1, 1)

            return 0

        lax.fori_loop(0, (NTC + 1) // 2, pair, 0)
        for st in range(2):
            for a in range(D // 8):
                pltpu.make_async_copy(
                    stage.at[st, a], out_hbm.at[0, a, wid], wsems[st]
                ).wait()

    out5 = body(idx_i32, token_table, pos_table)
    return out5.transpose(2, 4, 0, 1, 3).reshape(B, S, D)


# speedup vs baseline: 1.3749x; 1.3749x over previous
import functools

import jax
import jax.numpy as jnp
from jax import lax
from jax.experimental import pallas as pl
from jax.experimental.pallas import tpu as pltpu
from jax.experimental.pallas import tpu_sc as plsc

D = 32
SEQ = 200
BB = 128
TC = 8
NTC = SEQ // TC
VL = 16


def kernel(inputs, token_table, pos_table):
    B, S = inputs.shape
    V, d = token_table.shape
    assert S == SEQ and d == D and B % BB == 0

    info = plsc.get_sparse_core_info()
    NC, NS = info.num_cores, info.num_subcores
    NW = NC * NS
    assert B // BB == NW

    idx_i32 = inputs.astype(jnp.int32)

    mesh = plsc.VectorSubcoreMesh(core_axis_name="c", subcore_axis_name="s")

    @functools.partial(
        pl.kernel,
        mesh=mesh,
        compiler_params=pltpu.CompilerParams(use_tc_tiling_on_sc=False, needs_layout_passes=False),
        out_type=jax.ShapeDtypeStruct((SEQ, D // 8, NW, 8, BB), jnp.float32),
        scratch_types=[
            pltpu.VMEM((BB, SEQ), jnp.int32),
            pltpu.VMEM((2, TC, BB), jnp.int32),
            pltpu.VMEM((2, TC, BB, D), jnp.float32),
            pltpu.VMEM((2, D // 8, 8, BB), jnp.float32),
            pltpu.VMEM((SEQ, D), jnp.float32),
            [pltpu.SemaphoreType.DMA] * 2,
            [pltpu.SemaphoreType.DMA] * 2,
        ],
    )
    def body(idx_hbm, tok_hbm, pos_hbm, out_hbm,
             idx_all, idx_c, gbuf, stage, pos_v, gsems, wsems):
        wid = lax.axis_index("s") * NC + lax.axis_index("c")
        pltpu.sync_copy(pos_hbm, pos_v)
        pltpu.sync_copy(idx_hbm.at[pl.ds(wid * BB, BB)], idx_all)

        lanes = lax.iota(jnp.int32, VL)

        def build_and_fire(tc, par):
            for tl in range(TC):
                t = tc * TC + tl
                tvec = jnp.full((VL,), 0, jnp.int32) + t
                for v in range(BB // VL):
                    vals = plsc.load_gather(idx_all, [lanes + (v * VL), tvec])
                    idx_c[par, tl, pl.ds(v * VL, VL)] = vals
            for tl in range(TC):
                pltpu.async_copy(
                    tok_hbm.at[idx_c.at[par, tl]],
                    gbuf.at[par, tl],
                    gsems[par],
                )

        def drain_gather(par):
            for tl in range(TC):
                pltpu.make_async_copy(
                    tok_hbm.at[pl.ds(0, BB)], gbuf.at[par, tl], gsems[par]
                ).wait()

        def process_chunk(tc, par):
            def do_t(tl, st, first_ok):
                t = tc * TC + tl

                @pl.when(first_ok)
                def _():
                    for a in range(D // 8):
                        pltpu.make_async_copy(
                            stage.at[st, a], out_hbm.at[0, a, wid], wsems[st]
                        ).wait()

                p0 = pos_v[t, pl.ds(0, VL)]
                p1 = pos_v[t, pl.ds(VL, VL)]
                pvec = jnp.full((VL,), 0, jnp.int32) + par
                tvec = jnp.full((VL,), 0, jnp.int32) + tl
                for a in range(D // 8):
                    for v in range(BB // VL):
                        bvec = lanes + (v * VL)
                        gs = [
                            plsc.load_gather(
                                gbuf,
                                [pvec, tvec, bvec,
                                 jnp.full((VL,), 0, jnp.int32) + (a * 8 + cl)],
                            )
                            for cl in range(8)
                        ]
                        for cl in range(8):
                            c = a * 8 + cl
                            pv = (p0 if c < VL else p1)[c % VL]
                            stage[st, a, cl, pl.ds(v * VL, VL)] = gs[cl] + pv
                for a in range(D // 8):
                    pltpu.async_copy(
                        stage.at[st, a], out_hbm.at[t, a, wid], wsems[st]
                    )

            def two_t(tl2, _):
                ok = (tc > 0) | (tl2 >= 1)
                for par_t in range(2):
                    do_t(tl2 * 2 + par_t, par_t, ok)
                return 0

            lax.fori_loop(0, TC // 2, two_t, 0)

        build_and_fire(0, 0)

        def pair(p, _):
            c0 = 2 * p
            c1 = c0 + 1

            @pl.when(c1 < NTC)
            def _():
                build_and_fire(c1, 1)

            drain_gather(0)
            process_chunk(c0, 0)

            @pl.when(c1 < NTC)
            def _():
                @pl.when(c1 + 1 < NTC)
                def _():
                    build_and_fire(c1 + 1, 0)

                drain_gather(1)
                process_chunk(c1, 1)

            return 0

        lax.fori_loop(0, (NTC + 1) // 2, pair, 0)
        for st in range(2):
            for a in range(D // 8):
                pltpu.make_async_copy(
                    stage.at[st, a], out_hbm.at[0, a, wid], wsems[st]
                ).wait()

    out5 = body(idx_i32, token_table, pos_table)
    return out5.transpose(2, 4, 0, 1, 3).reshape(B, S, D)


# speedup vs baseline: 1.3843x; 1.0069x over previous
import functools

import jax
import jax.numpy as jnp
from jax import lax
from jax.experimental import pallas as pl
from jax.experimental.pallas import tpu as pltpu
from jax.experimental.pallas import tpu_sc as plsc

D = 32
SEQ = 200
BB = 128
TC = 8
NTC = SEQ // TC
VL = 16


def kernel(inputs, token_table, pos_table):
    B, S = inputs.shape
    V, d = token_table.shape
    assert S == SEQ and d == D and B % BB == 0

    info = plsc.get_sparse_core_info()
    NC, NS = info.num_cores, info.num_subcores
    NW = NC * NS
    assert B // BB == NW

    idx_i32 = inputs.astype(jnp.int32)

    mesh = plsc.VectorSubcoreMesh(core_axis_name="c", subcore_axis_name="s")

    NBLK = V // 128
    VTAIL = V - NBLK * 128
    VPAD = NBLK * 128 + ((VTAIL + 127) // 128) * 128
    BPW = (NBLK + NW - 1) // NW

    @functools.partial(
        pl.kernel,
        mesh=mesh,
        compiler_params=pltpu.CompilerParams(use_tc_tiling_on_sc=True, needs_layout_passes=False),
        out_type=jax.ShapeDtypeStruct((VPAD * D,), jnp.float32),
        scratch_types=[
            [pltpu.VMEM((D, 128), jnp.float32)] * 2,
            [pltpu.VMEM((128 * D,), jnp.float32)] * 2,
            [pltpu.SemaphoreType.DMA] * 2,
            [pltpu.SemaphoreType.DMA] * 2,
        ],
    )
    def untile(tokT_hbm, tail_hbm, lin_hbm, tins, routs, isems, osems):
        wid = lax.axis_index("s") * NC + lax.axis_index("c")
        j0 = wid * BPW
        nj = jnp.minimum(BPW, NBLK - j0)
        lanes16 = lax.iota(jnp.int32, 16)

        def fetch(jj, par):
            j = j0 + jj
            pltpu.async_copy(
                tokT_hbm.at[:, pl.ds(j * 128, 128)], tins[par], isems[par]
            )

        def transpose_block(jj, par):
            j = j0 + jj
            col = j * 128
            pltpu.make_async_copy(
                tokT_hbm.at[:, pl.ds(0, 128)], tins[par], isems[par]
            ).wait()

            def eight_rows(r8, _):
                gs = []
                for u in range(8):
                    rl = r8 * 8 + u
                    for h in range(2):
                        gs.append(
                            plsc.load_gather(
                                tins[par],
                                [lanes16 + (h * 16),
                                 jnp.full((16,), 0, jnp.int32) + rl],
                            )
                        )
                for u in range(8):
                    rl = r8 * 8 + u
                    for h in range(2):
                        routs[par][pl.ds(rl * D + h * 16, 16)] = gs[u * 2 + h]
                return 0

            lax.fori_loop(0, 16, eight_rows, 0)
            pltpu.async_copy(routs[par], lin_hbm.at[pl.ds(col * D, 128 * D)],
                             osems[par])

        def drain_out(par):
            pltpu.make_async_copy(
                routs[par], lin_hbm.at[pl.ds(0, 128 * D)], osems[par]
            ).wait()

        @pl.when(wid == 0)
        def _():
            pltpu.sync_copy(tail_hbm, lin_hbm.at[pl.ds(NBLK * 128 * D, VTAIL * D)])

        @pl.when(nj > 0)
        def _():
            fetch(0, 0)

        def pair(p, _):
            j0_, j1_ = 2 * p, 2 * p + 1

            @pl.when(j1_ < nj)
            def _():
                fetch(j1_, 1)

            @pl.when(j0_ < nj)
            def _():
                @pl.when(j0_ >= 2)
                def _():
                    drain_out(0)

                transpose_block(j0_, 0)

            @pl.when(j1_ < nj)
            def _():
                @pl.when(j1_ + 1 < nj)
                def _():
                    fetch(j1_ + 1, 0)

                @pl.when(j1_ >= 2)
                def _():
                    drain_out(1)

                transpose_block(j1_, 1)

            return 0

        lax.fori_loop(0, (BPW + 1) // 2, pair, 0)
        for par in range(2):
            @pl.when(nj > (1 - par))
            def _():
                drain_out(par)

    @functools.partial(
        pl.kernel,
        mesh=mesh,
        compiler_params=pltpu.CompilerParams(use_tc_tiling_on_sc=False, needs_layout_passes=False),
        out_type=jax.ShapeDtypeStruct((SEQ, D // 8, NW, 8, BB), jnp.float32),
        scratch_types=[
            pltpu.VMEM((BB, SEQ), jnp.int32),
            pltpu.VMEM((2, TC, BB), jnp.int32),
            pltpu.VMEM((2, TC, BB, D), jnp.float32),
            pltpu.VMEM((2, D // 8, 8, BB), jnp.float32),
            pltpu.VMEM((SEQ, D), jnp.float32),
            [pltpu.SemaphoreType.DMA] * 2,
            [pltpu.SemaphoreType.DMA] * 2,
        ],
    )
    def body(idx_hbm, tok_hbm, pos_hbm, out_hbm,
             idx_all, idx_c, gbuf, stage, pos_v, gsems, wsems):
        wid = lax.axis_index("s") * NC + lax.axis_index("c")
        pltpu.sync_copy(pos_hbm, pos_v)
        pltpu.sync_copy(idx_hbm.at[pl.ds(wid * BB, BB)], idx_all)

        lanes = lax.iota(jnp.int32, VL)

        def build_and_fire(tc, par):
            for tl in range(TC):
                t = tc * TC + tl
                tvec = jnp.full((VL,), 0, jnp.int32) + t
                for v in range(BB // VL):
                    vals = plsc.load_gather(idx_all, [lanes + (v * VL), tvec])
                    idx_c[par, tl, pl.ds(v * VL, VL)] = vals
            for tl in range(TC):
                pltpu.async_copy(
                    tok_hbm.at[idx_c.at[par, tl]],
                    gbuf.at[par, tl],
                    gsems[par],
                )

        def drain_gather(par):
            for tl in range(TC):
                pltpu.make_async_copy(
                    tok_hbm.at[pl.ds(0, BB)], gbuf.at[par, tl], gsems[par]
                ).wait()

        def process_chunk(tc, par):
            def do_t(tl, st, first_ok):
                t = tc * TC + tl

                @pl.when(first_ok)
                def _():
                    for a in range(D // 8):
                        pltpu.make_async_copy(
                            stage.at[st, a], out_hbm.at[0, a, wid], wsems[st]
                        ).wait()

                p0 = pos_v[t, pl.ds(0, VL)]
                p1 = pos_v[t, pl.ds(VL, VL)]
                pvec = jnp.full((VL,), 0, jnp.int32) + par
                tvec = jnp.full((VL,), 0, jnp.int32) + tl
                for a in range(D // 8):
                    for v in range(BB // VL):
                        bvec = lanes + (v * VL)
                        gs = [
                            plsc.load_gather(
                                gbuf,
                                [pvec, tvec, bvec,
                                 jnp.full((VL,), 0, jnp.int32) + (a * 8 + cl)],
                            )
                            for cl in range(8)
                        ]
                        for cl in range(8):
                            c = a * 8 + cl
                            pv = (p0 if c < VL else p1)[c % VL]
                            stage[st, a, cl, pl.ds(v * VL, VL)] = gs[cl] + pv
                for a in range(D // 8):
                    pltpu.async_copy(
                        stage.at[st, a], out_hbm.at[t, a, wid], wsems[st]
                    )

            def two_t(tl2, _):
                ok = (tc > 0) | (tl2 >= 1)
                for par_t in range(2):
                    do_t(tl2 * 2 + par_t, par_t, ok)
                return 0

            lax.fori_loop(0, TC // 2, two_t, 0)

        build_and_fire(0, 0)

        def pair(p, _):
            c0 = 2 * p
            c1 = c0 + 1

            @pl.when(c1 < NTC)
            def _():
                build_and_fire(c1, 1)

            drain_gather(0)
            process_chunk(c0, 0)

            @pl.when(c1 < NTC)
            def _():
                @pl.when(c1 + 1 < NTC)
                def _():
                    build_and_fire(c1 + 1, 0)

                drain_gather(1)
                process_chunk(c1, 1)

            return 0

        lax.fori_loop(0, (NTC + 1) // 2, pair, 0)
        for st in range(2):
            for a in range(D // 8):
                pltpu.make_async_copy(
                    stage.at[st, a], out_hbm.at[0, a, wid], wsems[st]
                ).wait()

    tail = lax.slice(token_table, (NBLK * 128, 0), (V, D)).reshape(-1)
    tok_lin = untile(token_table.T, tail).reshape(VPAD, D)
    out5 = body(idx_i32, tok_lin, pos_table)
    return out5.transpose(2, 4, 0, 1, 3).reshape(B, S, D)


# speedup vs baseline: 1.3857x; 1.0010x over previous
import functools

import jax
import jax.numpy as jnp
from jax import lax
from jax.experimental import pallas as pl
from jax.experimental.pallas import tpu as pltpu
from jax.experimental.pallas import tpu_sc as plsc

D = 32
SEQ = 200
BB = 128
TC = 8
NTC = SEQ // TC
VL = 16


def kernel(inputs, token_table, pos_table):
    B, S = inputs.shape
    V, d = token_table.shape
    assert S == SEQ and d == D and B % BB == 0

    info = plsc.get_sparse_core_info()
    NC, NS = info.num_cores, info.num_subcores
    NW = NC * NS
    assert B // BB == NW

    idx_i32 = inputs.astype(jnp.int32)

    mesh = plsc.VectorSubcoreMesh(core_axis_name="c", subcore_axis_name="s")

    NBLK = V // 128
    VTAIL = V - NBLK * 128
    VPAD = NBLK * 128 + ((VTAIL + 127) // 128) * 128
    W = 512
    NSUP = (NBLK * 128) // W
    BPW = (NSUP + NW - 1) // NW

    @functools.partial(
        pl.kernel,
        mesh=mesh,
        compiler_params=pltpu.CompilerParams(use_tc_tiling_on_sc=True, needs_layout_passes=False),
        out_type=jax.ShapeDtypeStruct((VPAD * D,), jnp.float32),
        scratch_types=[
            [pltpu.VMEM((D, W), jnp.float32)] * 2,
            [pltpu.VMEM((W * D,), jnp.float32)] * 2,
            [pltpu.SemaphoreType.DMA] * 2,
            [pltpu.SemaphoreType.DMA] * 2,
        ],
    )
    def untile(tokT_hbm, tail_hbm, lin_hbm, tins, routs, isems, osems):
        wid = lax.axis_index("s") * NC + lax.axis_index("c")
        j0 = wid * BPW
        nj = jnp.minimum(BPW, NSUP - j0)
        lanes16 = lax.iota(jnp.int32, 16)
        l32 = lanes16 * D

        def fetch(jj, par):
            j = j0 + jj
            pltpu.async_copy(
                tokT_hbm.at[:, pl.ds(j * W, W)], tins[par], isems[par]
            )

        def transpose_block(jj, par):
            j = j0 + jj
            col = j * W
            pltpu.make_async_copy(
                tokT_hbm.at[:, pl.ds(0, W)], tins[par], isems[par]
            ).wait()

            def vstep(v, _):
                idxbase = l32 + (v * (16 * D))
                xs = []
                for c in range(D):
                    xs.append(tins[par][c, pl.ds(v * 16, 16)])
                for c in range(D):
                    plsc.store_scatter(routs[par], [idxbase + c], xs[c])
                return 0

            lax.fori_loop(0, W // 16, vstep, 0)
            pltpu.async_copy(routs[par], lin_hbm.at[pl.ds(col * D, W * D)],
                             osems[par])

        def drain_out(par):
            pltpu.make_async_copy(
                routs[par], lin_hbm.at[pl.ds(0, W * D)], osems[par]
            ).wait()

        @pl.when(wid == 0)
        def _():
            pltpu.sync_copy(tail_hbm, lin_hbm.at[pl.ds(NBLK * 128 * D, VTAIL * D)])

        @pl.when(nj > 0)
        def _():
            fetch(0, 0)

        def pair(p, _):
            j0_, j1_ = 2 * p, 2 * p + 1

            @pl.when(j1_ < nj)
            def _():
                fetch(j1_, 1)

            @pl.when(j0_ < nj)
            def _():
                @pl.when(j0_ >= 2)
                def _():
                    drain_out(0)

                transpose_block(j0_, 0)

            @pl.when(j1_ < nj)
            def _():
                @pl.when(j1_ + 1 < nj)
                def _():
                    fetch(j1_ + 1, 0)

                @pl.when(j1_ >= 2)
                def _():
                    drain_out(1)

                transpose_block(j1_, 1)

            return 0

        lax.fori_loop(0, (BPW + 1) // 2, pair, 0)
        for par in range(2):
            @pl.when(nj > (1 - par))
            def _():
                drain_out(par)

    @functools.partial(
        pl.kernel,
        mesh=mesh,
        compiler_params=pltpu.CompilerParams(use_tc_tiling_on_sc=False, needs_layout_passes=False),
        out_type=jax.ShapeDtypeStruct((SEQ, D // 8, NW, 8, BB), jnp.float32),
        scratch_types=[
            pltpu.VMEM((BB, SEQ), jnp.int32),
            pltpu.VMEM((2, TC, BB), jnp.int32),
            pltpu.VMEM((2, TC, BB, D), jnp.float32),
            pltpu.VMEM((2, D // 8, 8, BB), jnp.float32),
            pltpu.VMEM((SEQ, D), jnp.float32),
            [pltpu.SemaphoreType.DMA] * 2,
            [pltpu.SemaphoreType.DMA] * 2,
        ],
    )
    def body(idx_hbm, tok_hbm, pos_hbm, out_hbm,
             idx_all, idx_c, gbuf, stage, pos_v, gsems, wsems):
        wid = lax.axis_index("s") * NC + lax.axis_index("c")
        pltpu.sync_copy(pos_hbm, pos_v)
        pltpu.sync_copy(idx_hbm.at[pl.ds(wid * BB, BB)], idx_all)

        lanes = lax.iota(jnp.int32, VL)

        def build_and_fire(tc, par):
            for tl in range(TC):
                t = tc * TC + tl
                tvec = jnp.full((VL,), 0, jnp.int32) + t
                for v in range(BB // VL):
                    vals = plsc.load_gather(idx_all, [lanes + (v * VL), tvec])
                    idx_c[par, tl, pl.ds(v * VL, VL)] = vals
            for tl in range(TC):
                pltpu.async_copy(
                    tok_hbm.at[idx_c.at[par, tl]],
                    gbuf.at[par, tl],
                    gsems[par],
                )

        def drain_gather(par):
            for tl in range(TC):
                pltpu.make_async_copy(
                    tok_hbm.at[pl.ds(0, BB)], gbuf.at[par, tl], gsems[par]
                ).wait()

        def process_chunk(tc, par):
            def do_t(tl, st, first_ok):
                t = tc * TC + tl

                @pl.when(first_ok)
                def _():
                    for a in range(D // 8):
                        pltpu.make_async_copy(
                            stage.at[st, a], out_hbm.at[0, a, wid], wsems[st]
                        ).wait()

                p0 = pos_v[t, pl.ds(0, VL)]
                p1 = pos_v[t, pl.ds(VL, VL)]
                pvec = jnp.full((VL,), 0, jnp.int32) + par
                tvec = jnp.full((VL,), 0, jnp.int32) + tl
                for a in range(D // 8):
                    for v in range(BB // VL):
                        bvec = lanes + (v * VL)
                        gs = [
                            plsc.load_gather(
                                gbuf,
                                [pvec, tvec, bvec,
                                 jnp.full((VL,), 0, jnp.int32) + (a * 8 + cl)],
                            )
                            for cl in range(8)
                        ]
                        for cl in range(8):
                            c = a * 8 + cl
                            pv = (p0 if c < VL else p1)[c % VL]
                            stage[st, a, cl, pl.ds(v * VL, VL)] = gs[cl] + pv
                for a in range(D // 8):
                    pltpu.async_copy(
                        stage.at[st, a], out_hbm.at[t, a, wid], wsems[st]
                    )

            def two_t(tl2, _):
                ok = (tc > 0) | (tl2 >= 1)
                for par_t in range(2):
                    do_t(tl2 * 2 + par_t, par_t, ok)
                return 0

            lax.fori_loop(0, TC // 2, two_t, 0)

        build_and_fire(0, 0)

        def pair(p, _):
            c0 = 2 * p
            c1 = c0 + 1

            @pl.when(c1 < NTC)
            def _():
                build_and_fire(c1, 1)

            drain_gather(0)
            process_chunk(c0, 0)

            @pl.when(c1 < NTC)
            def _():
                @pl.when(c1 + 1 < NTC)
                def _():
                    build_and_fire(c1 + 1, 0)

                drain_gather(1)
                process_chunk(c1, 1)

            return 0

        lax.fori_loop(0, (NTC + 1) // 2, pair, 0)
        for st in range(2):
            for a in range(D // 8):
                pltpu.make_async_copy(
                    stage.at[st, a], out_hbm.at[0, a, wid], wsems[st]
                ).wait()

    tail = lax.slice(token_table, (NBLK * 128, 0), (V, D)).reshape(-1)
    tok_lin = untile(token_table.T, tail).reshape(VPAD, D)
    out5 = body(idx_i32, tok_lin, pos_table)
    return out5.transpose(2, 4, 0, 1, 3).reshape(B, S, D)
